# baseline (speedup 1.0000x reference)
import functools

import jax
import jax.numpy as jnp
from jax import lax
from jax.experimental import pallas as pl
from jax.experimental.pallas import tpu as pltpu
from jax.experimental.pallas import tpu_sc as plsc

_NUM_CORES = 2
_NUM_SUBCORES = 16
_NW = _NUM_CORES * _NUM_SUBCORES
_B, _T, _C = 16384, 200, 6
_TOKENS = _B * _T
_TH = _T // 8
_BH = _B // 128
_BPW = _BH // _NW
_PLANE = _TOKENS
_INW = _BPW * 8 * 128
_OUTW = _INW * 2
_TROW = _BH * 2 * 128
_TABLE = 2560
_NPAIR = (_TH + 1) // 2


def _make_gather():
    mesh = plsc.VectorSubcoreMesh(core_axis_name="c", subcore_axis_name="s")

    @functools.partial(
        pl.kernel,
        mesh=mesh,
        out_type=jax.ShapeDtypeStruct((_TOKENS * 2,), jnp.float32),
        scratch_types=[
            pltpu.VMEM((_TABLE,), jnp.float32),
            pltpu.VMEM((_INW,), jnp.int32),
            pltpu.VMEM((_INW,), jnp.int32),
            pltpu.VMEM((_INW,), jnp.int32),
            pltpu.VMEM((_INW,), jnp.int32),
            pltpu.VMEM((_OUTW,), jnp.float32),
            pltpu.VMEM((_OUTW,), jnp.float32),
            pltpu.SemaphoreType.DMA,
            pltpu.SemaphoreType.DMA,
            pltpu.SemaphoreType.DMA,
            pltpu.SemaphoreType.DMA,
        ],
        compiler_params=pltpu.CompilerParams(needs_layout_passes=False),
    )
    def k(
        table_hbm, q_hbm, out_hbm,
        table_v, xb0, xb1, yb0, yb1, ob0, ob1,
        si0, si1, so0, so1,
    ):
        wid = lax.axis_index("s") * _NUM_CORES + lax.axis_index("c")
        in0 = wid * _INW
        ob_off = wid * _BPW * 256
        pltpu.sync_copy(table_hbm, table_v)

        xbufs = (xb0, xb1)
        ybufs = (yb0, yb1)
        obufs = (ob0, ob1)
        sin = (si0, si1)
        sout = (so0, so1)

        def in_x(th, b):
            src = q_hbm.at[pl.ds(_PLANE + th * (_BH * 1024) + in0, _INW)]
            return pltpu.make_async_copy(src, xbufs[b], sin[b])

        def in_y(th, b):
            src = q_hbm.at[pl.ds(2 * _PLANE + th * (_BH * 1024) + in0, _INW)]
            return pltpu.make_async_copy(src, ybufs[b], sin[b])

        def out_copy(th, tl, b):
            dst = out_hbm.at[pl.ds((th * 8 + tl) * _TROW + ob_off, _BPW * 256)]
            return pltpu.make_async_copy(
                obufs[b].at[pl.ds(tl * _BPW * 256, _BPW * 256)], dst, sout[b]
            )

        in_x(0, 0).start()
        in_y(0, 0).start()
        in_x(1, 1).start()
        in_y(1, 1).start()

        def pair_body(pair, _):
            for b in range(2):
                th = pair * 2 + b

                @pl.when(th < _TH)
                def _():
                    in_x(th, b).wait()
                    in_y(th, b).wait()

                    @pl.when(pair > 0)
                    def _():
                        for tl in range(8):
                            out_copy(th - 2, tl, b).wait()

                    qx = xbufs[b]
                    qy = ybufs[b]
                    ob = obufs[b]

                    @plsc.parallel_loop(0, 256, unroll=8)
                    def _(i):
                        tl = i >> 5
                        bhl = (i >> 3) & (_BPW - 1)
                        g = i & 7
                        in_base = bhl * 1024 + tl * 128 + g * 16
                        out_base = tl * (_BPW * 256) + bhl * 256 + g * 16
                        xi = qx[pl.ds(in_base, 16)]
                        ob[pl.ds(out_base, 16)] = plsc.load_gather(table_v, [xi])
                        yi = qy[pl.ds(in_base, 16)]
                        ob[pl.ds(out_base + 128, 16)] = (
                            plsc.load_gather(table_v, [yi])
                        )

                    for tl in range(8):
                        out_copy(th, tl, b).start()

                    @pl.when(th + 2 < _TH)
                    def _():
                        in_x(th + 2, b).start()
                        in_y(th + 2, b).start()

            return ()

        lax.fori_loop(0, _NPAIR, pair_body, ())
        for tl in range(8):
            out_copy(_TH - 2, tl, 1).wait()
            out_copy(_TH - 1, tl, 0).wait()

    return k


_gather = _make_gather()


def kernel(token_centers_lookup, query_tokens):
    qp = jnp.transpose(query_tokens, (2, 1, 0))
    qp = qp.reshape(_C, _TH, 8, _BH, 128)
    qp = jnp.transpose(qp, (0, 1, 3, 2, 4))
    qp = qp.reshape(-1)
    out = _gather(token_centers_lookup, qp)
    o = out.reshape(_T, _BH, 2, 128)
    o = jnp.transpose(o, (1, 3, 0, 2))
    return o.reshape(_B, _T, 2)


# speedup vs baseline: 1.0041x; 1.0041x over previous
import functools

import jax
import jax.numpy as jnp
from jax import lax
from jax.experimental import pallas as pl
from jax.experimental.pallas import tpu as pltpu
from jax.experimental.pallas import tpu_sc as plsc

_NUM_CORES = 2
_NUM_SUBCORES = 16
_NW = _NUM_CORES * _NUM_SUBCORES
_B, _T, _C = 16384, 200, 6
_TOKENS = _B * _T
_TH = _T // 8
_BH = _B // 128
_BPW = _BH // _NW
_PLANE = _TOKENS
_INW = _BPW * 8 * 128
_OUTW = _INW * 2
_TROW = _BH * 2 * 128
_TABLE = 2560
_NPAIR = (_TH + 1) // 2


def _make_gather():
    mesh = plsc.VectorSubcoreMesh(core_axis_name="c", subcore_axis_name="s")

    @functools.partial(
        pl.kernel,
        mesh=mesh,
        out_type=jax.ShapeDtypeStruct((_TOKENS * 2,), jnp.float32),
        scratch_types=[
            pltpu.VMEM((_TABLE,), jnp.float32),
            pltpu.VMEM((_INW,), jnp.int32),
            pltpu.VMEM((_INW,), jnp.int32),
            pltpu.VMEM((_INW,), jnp.int32),
            pltpu.VMEM((_INW,), jnp.int32),
            pltpu.VMEM((_OUTW,), jnp.float32),
            pltpu.VMEM((_OUTW,), jnp.float32),
            pltpu.SemaphoreType.DMA,
            pltpu.SemaphoreType.DMA,
            pltpu.SemaphoreType.DMA,
            pltpu.SemaphoreType.DMA,
        ],
        compiler_params=pltpu.CompilerParams(needs_layout_passes=False),
    )
    def k(
        table_hbm, q_hbm, out_hbm,
        table_v, xb0, xb1, yb0, yb1, ob0, ob1,
        si0, si1, so0, so1,
    ):
        wid = lax.axis_index("s") * _NUM_CORES + lax.axis_index("c")
        in0 = wid * _INW
        ob_off = wid * _BPW * 256
        pltpu.sync_copy(table_hbm, table_v)

        xbufs = (xb0, xb1)
        ybufs = (yb0, yb1)
        obufs = (ob0, ob1)
        sin = (si0, si1)
        sout = (so0, so1)

        def in_x(th, b):
            src = q_hbm.at[pl.ds(_PLANE + th * (_BH * 1024) + in0, _INW)]
            return pltpu.make_async_copy(src, xbufs[b], sin[b])

        def in_y(th, b):
            src = q_hbm.at[pl.ds(2 * _PLANE + th * (_BH * 1024) + in0, _INW)]
            return pltpu.make_async_copy(src, ybufs[b], sin[b])

        def out_copy(th, tl, b):
            dst = out_hbm.at[pl.ds((th * 8 + tl) * _TROW + ob_off, _BPW * 256)]
            return pltpu.make_async_copy(
                obufs[b].at[pl.ds(tl * _BPW * 256, _BPW * 256)], dst, sout[b]
            )

        in_x(0, 0).start()
        in_y(0, 0).start()
        in_x(1, 1).start()
        in_y(1, 1).start()

        def pair_body(pair, _):
            for b in range(2):
                th = pair * 2 + b

                @pl.when(th < _TH)
                def _():
                    in_x(th, b).wait()
                    in_y(th, b).wait()

                    @pl.when(pair > 0)
                    def _():
                        for tl in range(8):
                            out_copy(th - 2, tl, b).wait()

                    qx = xbufs[b]
                    qy = ybufs[b]
                    ob = obufs[b]

                    @plsc.parallel_loop(0, 256, unroll=4)
                    def _(i):
                        tl = i >> 5
                        bhl = (i >> 3) & (_BPW - 1)
                        g = i & 7
                        in_base = bhl * 1024 + tl * 128 + g * 16
                        out_base = tl * (_BPW * 256) + bhl * 256 + g * 16
                        xi = qx[pl.ds(in_base, 16)]
                        ob[pl.ds(out_base, 16)] = plsc.load_gather(table_v, [xi])
                        yi = qy[pl.ds(in_base, 16)]
                        ob[pl.ds(out_base + 128, 16)] = (
                            plsc.load_gather(table_v, [yi])
                        )

                    for tl in range(8):
                        out_copy(th, tl, b).start()

                    @pl.when(th + 2 < _TH)
                    def _():
                        in_x(th + 2, b).start()
                        in_y(th + 2, b).start()

            return ()

        lax.fori_loop(0, _NPAIR, pair_body, ())
        for tl in range(8):
            out_copy(_TH - 2, tl, 1).wait()
            out_copy(_TH - 1, tl, 0).wait()

    return k


_gather = _make_gather()


def kernel(token_centers_lookup, query_tokens):
    qp = jnp.transpose(query_tokens, (2, 1, 0))
    qp = qp.reshape(_C, _TH, 8, _BH, 128)
    qp = jnp.transpose(qp, (0, 1, 3, 2, 4))
    qp = qp.reshape(-1)
    out = _gather(token_centers_lookup, qp)
    o = out.reshape(_T, _BH, 2, 128)
    o = jnp.transpose(o, (1, 3, 0, 2))
    return o.reshape(_B, _T, 2)
